# Initial kernel scaffold; baseline (speedup 1.0000x reference)
import jax
import jax.numpy as jnp
from jax.experimental import pallas as pl


def kernel(in_shape, in_cond, params):
    raise NotImplementedError("write your pallas kernel here")


# speedup vs baseline: 11.4063x; 11.4063x over previous
import functools

import jax
import jax.numpy as jnp
from jax.experimental import pallas as pl
from jax.experimental.pallas import tpu as pltpu

_NPOINTS = [2048, 512, 128, 16]
_RADIUS = [0.02, 0.04, 0.06, 0.08]
_NSAMPLE = [32, 32, 16, 16]

_INTERPRET = False


def _fps_body(xyzT_ref, fidx_ref, dist_ref, *, npoint):
    B, N = dist_ref.shape
    xs = xyzT_ref[0]
    ys = xyzT_ref[1]
    zs = xyzT_ref[2]
    iota = jax.lax.broadcasted_iota(jnp.int32, (B, N), 1)
    iota_np = jax.lax.broadcasted_iota(jnp.int32, (B, npoint), 1)
    dist_ref[...] = jnp.full((B, N), 1e10, jnp.float32)

    def body(i, far):
        fidx_ref[...] = jnp.where(iota_np == i, far, fidx_ref[...])
        oh = iota == far
        cx = jnp.sum(jnp.where(oh, xs, 0.0), axis=1, keepdims=True)
        cy = jnp.sum(jnp.where(oh, ys, 0.0), axis=1, keepdims=True)
        cz = jnp.sum(jnp.where(oh, zs, 0.0), axis=1, keepdims=True)
        dx = xs - cx
        dy = ys - cy
        dz = zs - cz
        d = (dx * dx + dy * dy) + dz * dz
        dn = jnp.minimum(dist_ref[...], d)
        dist_ref[...] = dn
        m = jnp.max(dn, axis=1, keepdims=True)
        nf = jnp.min(jnp.where(dn == m, iota, N), axis=1, keepdims=True)
        return nf.astype(jnp.int32)

    jax.lax.fori_loop(0, npoint, body, jnp.zeros((B, 1), jnp.int32))


def _fps_call(xyzT, npoint):
    _, B, N = xyzT.shape
    return pl.pallas_call(
        functools.partial(_fps_body, npoint=npoint),
        out_shape=jax.ShapeDtypeStruct((B, npoint), jnp.int32),
        scratch_shapes=[pltpu.VMEM((B, N), jnp.float32)],
        interpret=_INTERPRET,
    )(xyzT)


def _bq_body(xyzT_ref, nxyz_ref, gidx_ref, *, r2, nsample):
    _, _, N = xyzT_ref.shape
    CB = nxyz_ref.shape[1]
    cx = nxyz_ref[0, :, 0:1]
    cy = nxyz_ref[0, :, 1:2]
    cz = nxyz_ref[0, :, 2:3]
    px = xyzT_ref[0, 0:1, :]
    py = xyzT_ref[0, 1:2, :]
    pz = xyzT_ref[0, 2:3, :]
    dx = cx - px
    dy = cy - py
    dz = cz - pz
    d2 = (dx * dx + dy * dy) + dz * dz
    cnt = (d2 < r2).astype(jnp.int32)
    sh = 1
    while sh < N:
        z = jnp.zeros((CB, sh), jnp.int32)
        cnt = cnt + jnp.concatenate([z, cnt[:, : N - sh]], axis=1)
        sh *= 2
    total = cnt[:, N - 1 : N]
    cols = [
        jnp.sum((cnt <= j).astype(jnp.int32), axis=1, keepdims=True)
        for j in range(nsample)
    ]
    g = jnp.concatenate(cols, axis=1)
    g0 = g[:, 0:1]
    jj = jax.lax.broadcasted_iota(jnp.int32, (CB, nsample), 1)
    gidx_ref[0] = jnp.where(jj < total, g, g0)


def _ball_query_call(xyzT_b, new_xyz, radius, nsample):
    B, _, N = xyzT_b.shape
    npnt = new_xyz.shape[1]
    CB = min(128, npnt)
    grid = (B, npnt // CB)
    r2 = float(radius) * float(radius)
    return pl.pallas_call(
        functools.partial(_bq_body, r2=r2, nsample=nsample),
        grid=grid,
        in_specs=[
            pl.BlockSpec((1, 3, N), lambda b, p: (b, 0, 0)),
            pl.BlockSpec((1, CB, 3), lambda b, p: (b, p, 0)),
        ],
        out_specs=pl.BlockSpec((1, CB, nsample), lambda b, p: (b, p, 0)),
        out_shape=jax.ShapeDtypeStruct((B, npnt, nsample), jnp.int32),
        interpret=_INTERPRET,
    )(xyzT_b, new_xyz)


def _pass1_body(xg_ref, cg_ref, w_ref, b_ref, y_ref, s_ref, q_ref, *, ns, cf):
    Mblk, D = xg_ref.shape
    G = Mblk // ns
    xg = xg_ref[...].reshape(G, ns, D)
    cg = cg_ref[...]
    rel = xg[:, :, 0:3] - cg[:, None, 0:3]
    if cf:
        xin = jnp.concatenate([rel, xg[:, :, 3 : 3 + cf]], axis=-1)
    else:
        xin = rel
    cin = 3 + cf
    y = (
        jnp.dot(
            xin.reshape(Mblk, cin),
            w_ref[...],
            preferred_element_type=jnp.float32,
        )
        + b_ref[...]
    )
    y_ref[...] = y

    @pl.when(pl.program_id(0) == 0)
    def _():
        s_ref[...] = jnp.zeros_like(s_ref)
        q_ref[...] = jnp.zeros_like(q_ref)

    s_ref[...] += jnp.sum(y, axis=0, keepdims=True)
    q_ref[...] += jnp.sum(y * y, axis=0, keepdims=True)


def _bn_scalars(s, q, g, be, m):
    mu = s * (1.0 / m)
    var = q * (1.0 / m) - mu * mu
    a = g * jax.lax.rsqrt(var + 1e-5)
    c = be - mu * a
    return a, c


def _pass2_body(yp_ref, s_in, q_in, g_ref, be_ref, w_ref, b_ref,
                y_ref, s_ref, q_ref, *, m):
    a, c = _bn_scalars(s_in[...], q_in[...], g_ref[...], be_ref[...], m)
    xn = jnp.maximum(yp_ref[...] * a + c, 0.0)
    y = jnp.dot(xn, w_ref[...], preferred_element_type=jnp.float32) + b_ref[...]
    y_ref[...] = y

    @pl.when(pl.program_id(0) == 0)
    def _():
        s_ref[...] = jnp.zeros_like(s_ref)
        q_ref[...] = jnp.zeros_like(q_ref)

    s_ref[...] += jnp.sum(y, axis=0, keepdims=True)
    q_ref[...] += jnp.sum(y * y, axis=0, keepdims=True)


def _pass3_body(yp_ref, s_in, q_in, g_ref, be_ref, w_ref, b_ref,
                pmax_ref, pmin_ref, s_ref, q_ref, *, m, ns):
    a, c = _bn_scalars(s_in[...], q_in[...], g_ref[...], be_ref[...], m)
    xn = jnp.maximum(yp_ref[...] * a + c, 0.0)
    y = jnp.dot(xn, w_ref[...], preferred_element_type=jnp.float32) + b_ref[...]
    Mblk, C = y.shape
    yg = y.reshape(Mblk // ns, ns, C)
    pmax_ref[...] = jnp.max(yg, axis=1)
    pmin_ref[...] = jnp.min(yg, axis=1)

    @pl.when(pl.program_id(0) == 0)
    def _():
        s_ref[...] = jnp.zeros_like(s_ref)
        q_ref[...] = jnp.zeros_like(q_ref)

    s_ref[...] += jnp.sum(y, axis=0, keepdims=True)
    q_ref[...] += jnp.sum(y * y, axis=0, keepdims=True)


def _pass4_body(pmax_ref, pmin_ref, s_in, q_in, g_ref, be_ref, o_ref, *, m):
    a, c = _bn_scalars(s_in[...], q_in[...], g_ref[...], be_ref[...], m)
    sel = jnp.where(a >= 0.0, pmax_ref[...], pmin_ref[...])
    o_ref[...] = jnp.maximum(sel * a + c, 0.0)


def _row(v):
    return v.reshape(1, -1)


def _mlp_stage(xg, cg, layer_params, ns, cf):
    M, D = xg.shape
    G = M // ns
    Mblk = min(8192, M)
    nblk = M // Mblk
    Gblk = Mblk // ns
    (w1, b1, g1, be1), (w2, b2, g2, be2), (w3, b3, g3, be3) = layer_params
    c1, c2, c3 = w1.shape[0], w2.shape[0], w3.shape[0]
    fm = float(M)

    stat = lambda C: jax.ShapeDtypeStruct((1, C), jnp.float32)
    stat_spec = lambda C: pl.BlockSpec((1, C), lambda i: (0, 0))

    y1, s1, q1 = pl.pallas_call(
        functools.partial(_pass1_body, ns=ns, cf=cf),
        grid=(nblk,),
        in_specs=[
            pl.BlockSpec((Mblk, D), lambda i: (i, 0)),
            pl.BlockSpec((Gblk, D), lambda i: (i, 0)),
            pl.BlockSpec(w1.T.shape, lambda i: (0, 0)),
            stat_spec(c1),
        ],
        out_specs=[
            pl.BlockSpec((Mblk, c1), lambda i: (i, 0)),
            stat_spec(c1),
            stat_spec(c1),
        ],
        out_shape=[
            jax.ShapeDtypeStruct((M, c1), jnp.float32),
            stat(c1),
            stat(c1),
        ],
        interpret=_INTERPRET,
    )(xg, cg, w1.T, _row(b1))

    y2, s2, q2 = pl.pallas_call(
        functools.partial(_pass2_body, m=fm),
        grid=(nblk,),
        in_specs=[
            pl.BlockSpec((Mblk, c1), lambda i: (i, 0)),
            stat_spec(c1),
            stat_spec(c1),
            stat_spec(c1),
            stat_spec(c1),
            pl.BlockSpec(w2.T.shape, lambda i: (0, 0)),
            stat_spec(c2),
        ],
        out_specs=[
            pl.BlockSpec((Mblk, c2), lambda i: (i, 0)),
            stat_spec(c2),
            stat_spec(c2),
        ],
        out_shape=[
            jax.ShapeDtypeStruct((M, c2), jnp.float32),
            stat(c2),
            stat(c2),
        ],
        interpret=_INTERPRET,
    )(y1, s1, q1, _row(g1), _row(be1), w2.T, _row(b2))

    pmax, pmin, s3, q3 = pl.pallas_call(
        functools.partial(_pass3_body, m=fm, ns=ns),
        grid=(nblk,),
        in_specs=[
            pl.BlockSpec((Mblk, c2), lambda i: (i, 0)),
            stat_spec(c2),
            stat_spec(c2),
            stat_spec(c2),
            stat_spec(c2),
            pl.BlockSpec(w3.T.shape, lambda i: (0, 0)),
            stat_spec(c3),
        ],
        out_specs=[
            pl.BlockSpec((Gblk, c3), lambda i: (i, 0)),
            pl.BlockSpec((Gblk, c3), lambda i: (i, 0)),
            stat_spec(c3),
            stat_spec(c3),
        ],
        out_shape=[
            jax.ShapeDtypeStruct((G, c3), jnp.float32),
            jax.ShapeDtypeStruct((G, c3), jnp.float32),
            stat(c3),
            stat(c3),
        ],
        interpret=_INTERPRET,
    )(y2, s2, q2, _row(g2), _row(be2), w3.T, _row(b3))

    feats_rows = pl.pallas_call(
        functools.partial(_pass4_body, m=fm),
        out_shape=jax.ShapeDtypeStruct((G, c3), jnp.float32),
        interpret=_INTERPRET,
    )(pmax, pmin, s3, q3, _row(g3), _row(be3))
    return feats_rows


def _gather_rows(table, idx):
    return table[idx]


def _pad_rows(x, d):
    if x.shape[-1] == d:
        return x
    return jnp.pad(x, ((0, 0), (0, d - x.shape[-1])))


def kernel(in_shape, in_cond, params):
    del in_cond
    B = in_shape.shape[0]
    xyz = in_shape[..., 0:3].reshape(B, -1, 3)
    N0 = xyz.shape[1]

    l_xyz = [xyz]
    l_feats = []

    xyz_rows = xyz.reshape(B * N0, 3)
    feat_rows = None
    n_cur = N0

    for k in range(4):
        npnt = _NPOINTS[k]
        ns = _NSAMPLE[k]
        cf = 0 if feat_rows is None else feat_rows.shape[1]
        d_tab = -(-(3 + cf) // 16) * 16

        if cf:
            table = _pad_rows(
                jnp.concatenate([xyz_rows, feat_rows], axis=1), d_tab
            )
        else:
            table = _pad_rows(xyz_rows, d_tab)

        xyzT = jnp.transpose(xyz_rows.reshape(B, n_cur, 3), (2, 0, 1))
        fidx = _fps_call(xyzT, npnt)

        base = (jnp.arange(B, dtype=jnp.int32) * n_cur)[:, None]
        cg_idx = (fidx + base).reshape(-1)
        cg = _gather_rows(table, cg_idx)
        new_xyz = cg[:, 0:3].reshape(B, npnt, 3)
        l_xyz.append(new_xyz)

        xyzT_b = jnp.transpose(xyz_rows.reshape(B, n_cur, 3), (0, 2, 1))
        gidx = _ball_query_call(xyzT_b, new_xyz, _RADIUS[k], ns)

        gg_idx = (gidx + base[:, :, None]).reshape(-1)
        xg = _gather_rows(table, gg_idx)

        feat_rows = _mlp_stage(xg, cg, params[k], ns, cf)
        l_feats.append(
            jnp.transpose(feat_rows.reshape(B, npnt, -1), (0, 2, 1))
        )
        xyz_rows = cg[:, 0:3]
        n_cur = npnt

    return (tuple(l_xyz), tuple(l_feats))
